# Initial kernel scaffold; baseline (speedup 1.0000x reference)
import jax
import jax.numpy as jnp
from jax.experimental import pallas as pl


def kernel(x, x_mask, neighbour_h, neighbour_c, W_iou, b_iou, W_f, b_f, U_iou, bU_iou, U_f, bU_f):
    raise NotImplementedError("write your pallas kernel here")


# speedup vs baseline: 1.1443x; 1.1443x over previous
import functools

import jax
import jax.numpy as jnp
from jax.experimental import pallas as pl
from jax.experimental.pallas import tpu as pltpu

_BN = 512
_BK = 512


def _lstm_kernel(hm_ref, uc_ref, x_ref, wc_ref, nc_ref, mask_ref,
                 bx_ref, bu_ref, h_out, c_out, acc_ref, *, nk, n_ch, h_size):
    k = pl.program_id(1)

    @pl.when(k == 0)
    def _():
        acc_ref[...] = jnp.zeros_like(acc_ref)

    acc_ref[...] += jnp.dot(hm_ref[...].astype(jnp.bfloat16), uc_ref[...],
                            preferred_element_type=jnp.float32)

    @pl.when(k == nk - 1)
    def _():
        mask = mask_ref[...]
        xg = jnp.dot(x_ref[...].astype(jnp.bfloat16), wc_ref[...],
                     preferred_element_type=jnp.float32)
        xb = (xg + bx_ref[...]) * mask
        acc = acc_ref[...] + bu_ref[...]
        f_in = xb[:, :h_size]
        c_aggr = jnp.zeros((acc.shape[0], h_size), dtype=jnp.float32)
        for ch in range(n_ch):
            lo = ch * h_size
            f = jax.nn.sigmoid(acc[:, lo:lo + h_size] + f_in)
            c_aggr += f * nc_ref[:, lo:lo + h_size]
        base = n_ch * h_size
        iou = acc[:, base:] + xb[:, h_size:]
        i_g = jax.nn.sigmoid(iou[:, :h_size])
        o_g = jax.nn.sigmoid(iou[:, h_size:2 * h_size])
        u_g = jnp.tanh(iou[:, 2 * h_size:])
        c = i_g * u_g + c_aggr
        h_out[...] = o_g * jnp.tanh(c)
        c_out[...] = c


@functools.partial(jax.jit, static_argnums=())
def kernel(x, x_mask, neighbour_h, neighbour_c, W_iou, b_iou, W_f, b_f,
           U_iou, bU_iou, U_f, bU_f):
    n, n_ch, h_size = neighbour_h.shape
    x_size = x.shape[1]
    kdim = n_ch * h_size
    ncols = kdim + 3 * h_size

    hm = neighbour_h.reshape(n, kdim)
    nc = neighbour_c.reshape(n, kdim)
    u_cat = jnp.concatenate(
        [U_f[:, :kdim], U_iou], axis=1).astype(jnp.bfloat16)
    w_cat = jnp.concatenate([W_f, W_iou], axis=1).astype(jnp.bfloat16)
    bx = jnp.concatenate([b_f, b_iou])[None, :]
    bu = jnp.concatenate([bU_f[:kdim], bU_iou])[None, :]
    mask = x_mask[:, None]

    ni = pl.cdiv(n, _BN)
    nk = kdim // _BK

    grid = (ni, nk)
    out_shape = (
        jax.ShapeDtypeStruct((n, h_size), jnp.float32),
        jax.ShapeDtypeStruct((n, h_size), jnp.float32),
    )
    h_out, c_out = pl.pallas_call(
        functools.partial(_lstm_kernel, nk=nk, n_ch=n_ch, h_size=h_size),
        grid=grid,
        in_specs=[
            pl.BlockSpec((_BN, _BK), lambda i, k: (i, k)),
            pl.BlockSpec((_BK, ncols), lambda i, k: (k, 0)),
            pl.BlockSpec((_BN, x_size), lambda i, k: (i, 0)),
            pl.BlockSpec((x_size, 4 * h_size), lambda i, k: (0, 0)),
            pl.BlockSpec((_BN, kdim), lambda i, k: (i, 0)),
            pl.BlockSpec((_BN, 1), lambda i, k: (i, 0)),
            pl.BlockSpec((1, 4 * h_size), lambda i, k: (0, 0)),
            pl.BlockSpec((1, ncols), lambda i, k: (0, 0)),
        ],
        out_specs=(
            pl.BlockSpec((_BN, h_size), lambda i, k: (i, 0)),
            pl.BlockSpec((_BN, h_size), lambda i, k: (i, 0)),
        ),
        out_shape=out_shape,
        scratch_shapes=[pltpu.VMEM((_BN, ncols), jnp.float32)],
        compiler_params=pltpu.CompilerParams(
            dimension_semantics=("arbitrary", "arbitrary"),
        ),
    )(hm, u_cat, x, w_cat, nc, mask, bx, bu)
    return h_out, c_out


# speedup vs baseline: 1.2377x; 1.0816x over previous
import functools

import jax
import jax.numpy as jnp
from jax.experimental import pallas as pl
from jax.experimental.pallas import tpu as pltpu

_BN = 512
_CPK = 2


def _lstm_kernel(h3_ref, uf_ref, ui_ref, x_ref, wf_ref, wi_ref, nc_ref,
                 mask_ref, bf_ref, bi_ref, buf_ref, bui_ref,
                 h_out, c_out, accf_ref, acci_ref, *, nk, n_ch, h_size):
    k = pl.program_id(1)

    @pl.when(k == 0)
    def _():
        accf_ref[...] = jnp.zeros_like(accf_ref)
        acci_ref[...] = jnp.zeros_like(acci_ref)

    for d in range(_CPK):
        ch = k * _CPK + d
        hb = h3_ref[:, ch, :].astype(jnp.bfloat16)
        uf = uf_ref[d * h_size:(d + 1) * h_size, :]
        ui = ui_ref[d * h_size:(d + 1) * h_size, :]
        accf_ref[...] += jnp.dot(hb, uf, preferred_element_type=jnp.float32)
        acci_ref[...] += jnp.dot(hb, ui, preferred_element_type=jnp.float32)

    @pl.when(k == nk - 1)
    def _():
        mask = mask_ref[...]
        xb = x_ref[...].astype(jnp.bfloat16)
        f_in = (jnp.dot(xb, wf_ref[...], preferred_element_type=jnp.float32)
                + bf_ref[...]) * mask
        x_iou = (jnp.dot(xb, wi_ref[...], preferred_element_type=jnp.float32)
                 + bi_ref[...]) * mask
        accf = accf_ref[...] + buf_ref[...]
        c_aggr = jnp.zeros((accf.shape[0], h_size), dtype=jnp.float32)
        for ch in range(n_ch):
            lo = ch * h_size
            f = jax.nn.sigmoid(accf[:, lo:lo + h_size] + f_in)
            c_aggr += f * nc_ref[:, ch, :]
        iou = acci_ref[...] + bui_ref[...] + x_iou
        i_g = jax.nn.sigmoid(iou[:, :h_size])
        o_g = jax.nn.sigmoid(iou[:, h_size:2 * h_size])
        u_g = jnp.tanh(iou[:, 2 * h_size:])
        c = i_g * u_g + c_aggr
        h_out[...] = o_g * jnp.tanh(c)
        c_out[...] = c


def kernel(x, x_mask, neighbour_h, neighbour_c, W_iou, b_iou, W_f, b_f,
           U_iou, bU_iou, U_f, bU_f):
    n, n_ch, h_size = neighbour_h.shape
    x_size = x.shape[1]
    kdim = n_ch * h_size
    bk = _CPK * h_size
    nk = n_ch // _CPK
    ni = pl.cdiv(n, _BN)

    uf16 = U_f[:, :kdim].astype(jnp.bfloat16)
    ui16 = U_iou.astype(jnp.bfloat16)
    wf16 = W_f.astype(jnp.bfloat16)
    wi16 = W_iou.astype(jnp.bfloat16)
    mask = x_mask[:, None]
    bf2 = b_f[None, :]
    bi2 = b_iou[None, :]
    buf2 = bU_f[None, :kdim]
    bui2 = bU_iou[None, :]

    out_shape = (
        jax.ShapeDtypeStruct((n, h_size), jnp.float32),
        jax.ShapeDtypeStruct((n, h_size), jnp.float32),
    )
    h_out, c_out = pl.pallas_call(
        functools.partial(_lstm_kernel, nk=nk, n_ch=n_ch, h_size=h_size),
        grid=(ni, nk),
        in_specs=[
            pl.BlockSpec((_BN, n_ch, h_size), lambda i, k: (i, 0, 0)),
            pl.BlockSpec((bk, kdim), lambda i, k: (k, 0)),
            pl.BlockSpec((bk, 3 * h_size), lambda i, k: (k, 0)),
            pl.BlockSpec((_BN, x_size), lambda i, k: (i, 0)),
            pl.BlockSpec((x_size, h_size), lambda i, k: (0, 0)),
            pl.BlockSpec((x_size, 3 * h_size), lambda i, k: (0, 0)),
            pl.BlockSpec((_BN, n_ch, h_size), lambda i, k: (i, 0, 0)),
            pl.BlockSpec((_BN, 1), lambda i, k: (i, 0)),
            pl.BlockSpec((1, h_size), lambda i, k: (0, 0)),
            pl.BlockSpec((1, 3 * h_size), lambda i, k: (0, 0)),
            pl.BlockSpec((1, kdim), lambda i, k: (0, 0)),
            pl.BlockSpec((1, 3 * h_size), lambda i, k: (0, 0)),
        ],
        out_specs=(
            pl.BlockSpec((_BN, h_size), lambda i, k: (i, 0)),
            pl.BlockSpec((_BN, h_size), lambda i, k: (i, 0)),
        ),
        out_shape=out_shape,
        scratch_shapes=[
            pltpu.VMEM((_BN, kdim), jnp.float32),
            pltpu.VMEM((_BN, 3 * h_size), jnp.float32),
        ],
        compiler_params=pltpu.CompilerParams(
            dimension_semantics=("arbitrary", "arbitrary"),
        ),
    )(neighbour_h, uf16, ui16, x, wf16, wi16, neighbour_c, mask,
      bf2, bi2, buf2, bui2)
    return h_out, c_out
